# baseline (speedup 1.0000x reference)
import functools

import jax
import jax.numpy as jnp
from jax import lax
from jax.experimental import pallas as pl
from jax.experimental.pallas import tpu as pltpu
from jax.experimental.pallas import tpu_sc as plsc

_TIME = 288
_WPAD = 8
_CTROW = _TIME * _WPAD
_B, _T, _N, _F = 64, 12, 2048, 128
_NC, _NS, _L = 2, 16, 16
_NW = _NC * _NS
_FPW = _F // _NW
_BPS = _B // _NS
_CHUNKS = _N // _L


def _body(xs_hbm, td_hbm, tw_hbm, out_hbm,
          td_v, tw_v, ct_v, fbuf_v, wbuf_v, idx_v,
          ib0, ib1, or0, or1, idx_sh, sem_in, sem_out):
    cid = lax.axis_index("c")
    sid = lax.axis_index("s")
    wid = sid * _NC + cid
    f0 = wid * _FPW
    iota = lax.iota(jnp.int32, _L)

    with jax.named_scope("tables_in"):
        pltpu.sync_copy(td_hbm, td_v)
        pltpu.sync_copy(tw_hbm, tw_v)

    def _build(c, carry):
        ivec = c * _L + iota
        dvec = lax.shift_right_logical(ivec, 3)
        wvec = jnp.minimum(ivec & (_WPAD - 1), 6)
        for fl in range(_FPW):
            fvec = iota * 0 + (f0 + fl)
            tdcol = plsc.load_gather(td_v, [dvec, fvec])
            twcol = plsc.load_gather(tw_v, [wvec, fvec])
            ct_v[pl.ds(fl * _CTROW + c * _L, _L)] = tdcol + twcol
        return carry

    with jax.named_scope("ct_build"):
        lax.fori_loop(0, _CTROW // _L, _build, 0)

    for i in range(_BPS):
        bb = sid * _BPS + i
        pltpu.sync_copy(xs_hbm.at[bb, 0], fbuf_v)
        pltpu.sync_copy(xs_hbm.at[bb, 1], wbuf_v)

        @plsc.parallel_loop(0, _CHUNKS, unroll=2)
        def _decode(c):
            frac = fbuf_v[pl.ds(c * _L, _L)]
            wraw = wbuf_v[pl.ds(c * _L, _L)]
            d = (frac * float(_TIME)).astype(jnp.int32)
            d = jnp.minimum(jnp.maximum(d, 0), _TIME - 1)
            w = wraw.astype(jnp.int32)
            w = jnp.minimum(jnp.maximum(w, 0), 6)
            idx_v[pl.ds(c * _L, _L)] = d * _WPAD + w
        pltpu.sync_copy(idx_v, idx_sh.at[bb])

    with jax.named_scope("phaseA_barrier"):
        plsc.subcore_barrier()

    ibufs = (ib0, ib1)
    obufs = (or0, or1)

    def _in_copy(b, buf):
        return pltpu.make_async_copy(idx_sh.at[b], buf, sem_in)

    def _out_copy(b, buf):
        return pltpu.make_async_copy(buf, out_hbm.at[b, pl.ds(f0, _FPW)],
                                     sem_out)

    _in_copy(0, ibufs[0]).start()

    def _pair(g, carry):
        for j in range(2):
            b = g * 2 + j
            ib = ibufs[j]
            orow = obufs[j]
            _in_copy(b, ib).wait()

            @pl.when(b + 1 < _B)
            def _prefetch():
                _in_copy(b + 1, ibufs[1 - j]).start()

            @pl.when(b >= 2)
            def _drain():
                _out_copy(b - 2, orow).wait()

            @plsc.parallel_loop(0, _CHUNKS, unroll=8)
            def _chunk(c):
                base = c * _L
                cidx = ib[pl.ds(base, _L)]
                for fl in range(_FPW):
                    v = plsc.load_gather(
                        ct_v.at[pl.ds(fl * _CTROW, _CTROW)], [cidx])
                    orow[fl, pl.ds(base, _L)] = v
            _out_copy(b, orow).start()
        return carry

    with jax.named_scope("phaseB"):
        lax.fori_loop(0, _B // 2, _pair, 0)
        _out_copy(_B - 2, obufs[0]).wait()
        _out_copy(_B - 1, obufs[1]).wait()


@jax.jit
def _run(xs, time_day_flat, time_week_flat):
    mesh = plsc.VectorSubcoreMesh(core_axis_name="c", subcore_axis_name="s")
    k = functools.partial(
        pl.kernel,
        out_type=jax.ShapeDtypeStruct((_B, _F, _N), jnp.float32),
        mesh=mesh,
        compiler_params=pltpu.CompilerParams(needs_layout_passes=False),
        scratch_types=[
            pltpu.VMEM((_TIME, _F), jnp.float32),
            pltpu.VMEM((7, _F), jnp.float32),
            pltpu.VMEM((_FPW * _CTROW,), jnp.float32),
            pltpu.VMEM((_N,), jnp.float32),
            pltpu.VMEM((_N,), jnp.float32),
            pltpu.VMEM((_N,), jnp.int32),
            pltpu.VMEM((_N,), jnp.int32),
            pltpu.VMEM((_N,), jnp.int32),
            pltpu.VMEM((_FPW, _N), jnp.float32),
            pltpu.VMEM((_FPW, _N), jnp.float32),
            pltpu.VMEM_SHARED((_B, _N), jnp.int32),
            pltpu.SemaphoreType.DMA,
            pltpu.SemaphoreType.DMA,
        ],
    )(_body)
    return k(xs, time_day_flat, time_week_flat)


def kernel(x, time_day, time_week):
    xs = x[:, -1, :, 1:3].transpose(0, 2, 1)
    out = _run(xs, time_day, time_week)
    return out[..., None]


# speedup vs baseline: 1.6605x; 1.6605x over previous
import functools

import jax
import jax.numpy as jnp
from jax import lax
from jax.experimental import pallas as pl
from jax.experimental.pallas import tpu as pltpu
from jax.experimental.pallas import tpu_sc as plsc

_TIME = 288
_NWEEK = 7
_CTROW = 2048
_B, _T, _N, _F = 64, 12, 2048, 128
_NC, _NS, _L = 2, 16, 16
_NW = _NC * _NS
_FPW = _F // _NW
_BPS = _B // _NS
_CHUNKS = _N // _L


def _body(xs_hbm, td_hbm, tw_hbm, out_hbm,
          td_v, tw_v, ct_v, fbuf_v, wbuf_v, idx_v,
          ib0, ib1, or0, or1, idx_sh, sem_in, sem_out):
    cid = lax.axis_index("c")
    sid = lax.axis_index("s")
    wid = sid * _NC + cid
    f0 = wid * _FPW
    iota = lax.iota(jnp.int32, _L)

    with jax.named_scope("tables_in"):
        pltpu.sync_copy(td_hbm, td_v)
        pltpu.sync_copy(tw_hbm, tw_v)

    def _build(c, carry):
        dvec = c * _L + iota
        for fl in range(_FPW):
            fvec = iota * 0 + (f0 + fl)
            tdcol = plsc.load_gather(td_v, [dvec, fvec])
            for w in range(_NWEEK):
                twcol = plsc.load_gather(tw_v, [iota * 0 + w, fvec])
                ct_v[pl.ds(fl * _CTROW + w * _TIME + c * _L, _L)] = (
                    tdcol + twcol)
        return carry

    with jax.named_scope("ct_build"):
        lax.fori_loop(0, _TIME // _L, _build, 0)

    for i in range(_BPS):
        bb = sid * _BPS + i
        pltpu.sync_copy(xs_hbm.at[bb, 0], fbuf_v)
        pltpu.sync_copy(xs_hbm.at[bb, 1], wbuf_v)

        @plsc.parallel_loop(0, _CHUNKS, unroll=2)
        def _decode(c):
            frac = fbuf_v[pl.ds(c * _L, _L)]
            wraw = wbuf_v[pl.ds(c * _L, _L)]
            d = (frac * float(_TIME)).astype(jnp.int32)
            d = jnp.minimum(jnp.maximum(d, 0), _TIME - 1)
            w = wraw.astype(jnp.int32)
            w = jnp.minimum(jnp.maximum(w, 0), 6)
            idx_v[pl.ds(c * _L, _L)] = w * _TIME + d
        pltpu.sync_copy(idx_v, idx_sh.at[bb])

    with jax.named_scope("phaseA_barrier"):
        plsc.subcore_barrier()

    ibufs = (ib0, ib1)
    obufs = (or0, or1)

    def _in_copy(b, buf):
        return pltpu.make_async_copy(idx_sh.at[b], buf, sem_in)

    def _out_copy(b, buf):
        return pltpu.make_async_copy(buf, out_hbm.at[b, pl.ds(f0, _FPW)],
                                     sem_out)

    _in_copy(0, ibufs[0]).start()

    def _pair(g, carry):
        for j in range(2):
            b = g * 2 + j
            ib = ibufs[j]
            orow = obufs[j]
            _in_copy(b, ib).wait()

            @pl.when(b + 1 < _B)
            def _prefetch():
                _in_copy(b + 1, ibufs[1 - j]).start()

            @pl.when(b >= 2)
            def _drain():
                _out_copy(b - 2, orow).wait()

            @plsc.parallel_loop(0, _CHUNKS, unroll=8)
            def _chunk(c):
                base = c * _L
                cidx = ib[pl.ds(base, _L)]
                for fl in range(_FPW):
                    v = plsc.load_gather(
                        ct_v.at[pl.ds(fl * _CTROW, _CTROW)], [cidx])
                    orow[fl, pl.ds(base, _L)] = v
            _out_copy(b, orow).start()
        return carry

    with jax.named_scope("phaseB"):
        lax.fori_loop(0, _B // 2, _pair, 0)
        _out_copy(_B - 2, obufs[0]).wait()
        _out_copy(_B - 1, obufs[1]).wait()


@jax.jit
def _run(xs, time_day_flat, time_week_flat):
    mesh = plsc.VectorSubcoreMesh(core_axis_name="c", subcore_axis_name="s")
    k = functools.partial(
        pl.kernel,
        out_type=jax.ShapeDtypeStruct((_B, _F, _N), jnp.float32),
        mesh=mesh,
        compiler_params=pltpu.CompilerParams(needs_layout_passes=False),
        scratch_types=[
            pltpu.VMEM((_TIME, _F), jnp.float32),
            pltpu.VMEM((7, _F), jnp.float32),
            pltpu.VMEM((_FPW * _CTROW,), jnp.float32),
            pltpu.VMEM((_N,), jnp.float32),
            pltpu.VMEM((_N,), jnp.float32),
            pltpu.VMEM((_N,), jnp.int32),
            pltpu.VMEM((_N,), jnp.int32),
            pltpu.VMEM((_N,), jnp.int32),
            pltpu.VMEM((_FPW, _N), jnp.float32),
            pltpu.VMEM((_FPW, _N), jnp.float32),
            pltpu.VMEM_SHARED((_B, _N), jnp.int32),
            pltpu.SemaphoreType.DMA,
            pltpu.SemaphoreType.DMA,
        ],
    )(_body)
    return k(xs, time_day_flat, time_week_flat)


def kernel(x, time_day, time_week):
    xs = x[:, -1, :, 1:3].transpose(0, 2, 1)
    out = _run(xs, time_day, time_week)
    return out[..., None]
